# baseline (speedup 1.0000x reference)
import functools

import jax
import jax.numpy as jnp
from jax import lax
from jax.experimental import pallas as pl
from jax.experimental.pallas import tpu as pltpu
from jax.experimental.pallas import tpu_sc as plsc

N = 10000
D = 128
E = 320000

NC, NS = 2, 16
NW = NC * NS
CHUNK = 128
K_DEG = (32, 128)
K_SEG = (160, 0)
CH_TOT = NS * (K_SEG[0] + K_SEG[1])
E_PAD = CH_TOT * CHUNK
N_PAD = 10240
RPS = N_PAD // NS
SUPER = 16

_mesh = plsc.VectorSubcoreMesh(
    core_axis_name="c", subcore_axis_name="s", num_cores=NC, num_subcores=NS
)


def _core_span(c, s, k_pair):
    k0, k1 = k_pair
    base = jnp.where(c == 0, s * k0, NS * k0 + s * k1)
    nsup = jnp.where(c == 0, k0 // SUPER, k1 // SUPER)
    return base, nsup


def _fill(buf, value):
    v = jnp.full((16,), value, jnp.float32)

    @pl.loop(0, CHUNK)
    def _rows(r):
        for b in range(D // 16):
            buf[r, pl.ds(b * 16, 16)] = v


@functools.partial(
    pl.kernel,
    out_type=jax.ShapeDtypeStruct((NC, N_PAD, D), jnp.float32),
    mesh=_mesh,
    scratch_types=[
        pltpu.VMEM((SUPER, 2, CHUNK), jnp.int32),
        pltpu.VMEM((CHUNK, D), jnp.float32),
        pltpu.VMEM_SHARED((N_PAD, D), jnp.float32),
    ],
)
def _sc_degree(edges, out, idx_v, ones_v, acc):
    c = lax.axis_index("c")
    s = lax.axis_index("s")
    base, nsup = _core_span(c, s, K_DEG)
    _fill(ones_v, 0.0)
    for k in range(RPS // CHUNK):
        pltpu.sync_copy(ones_v, acc.at[pl.ds(s * RPS + k * CHUNK, CHUNK)])
    _fill(ones_v, 1.0)
    plsc.subcore_barrier()

    @pl.loop(0, nsup)
    def _super(sb):
        pltpu.sync_copy(edges.at[pl.ds(base + sb * SUPER, SUPER)], idx_v)

        @pl.loop(0, SUPER)
        def _count(j):
            pltpu.sync_copy(ones_v, acc.at[idx_v.at[j, 1]], add=True)

    plsc.subcore_barrier()
    for k in range(RPS // CHUNK):
        pltpu.sync_copy(
            acc.at[pl.ds(s * RPS + k * CHUNK, CHUNK)],
            out.at[c].at[pl.ds(s * RPS + k * CHUNK, CHUNK)],
        )


@functools.partial(
    pl.kernel,
    out_type=jax.ShapeDtypeStruct((NC, N_PAD, D), jnp.float32),
    mesh=_mesh,
    scratch_types=[
        pltpu.VMEM((SUPER, 2, CHUNK), jnp.int32),
        pltpu.VMEM((CHUNK, D), jnp.float32),
        pltpu.VMEM((CHUNK, D), jnp.float32),
        pltpu.VMEM_SHARED((N_PAD, D), jnp.float32),
        pltpu.SemaphoreType.DMA,
        pltpu.SemaphoreType.DMA,
    ],
)
def _sc_segsum(hp, edges, zeros_hbm, out, idx_v, buf0, buf1, acc, sem0, sem1):
    c = lax.axis_index("c")
    s = lax.axis_index("s")
    base, nsup = _core_span(c, s, K_SEG)
    pltpu.sync_copy(zeros_hbm, buf0)
    for k in range(RPS // CHUNK):
        pltpu.sync_copy(buf0, acc.at[pl.ds(s * RPS + k * CHUNK, CHUNK)])
    plsc.subcore_barrier()

    bufs = (buf0, buf1)
    sems = (sem0, sem1)

    @pl.loop(0, nsup)
    def _super(sb):
        pltpu.sync_copy(edges.at[pl.ds(base + sb * SUPER, SUPER)], idx_v)
        pltpu.async_copy(hp.at[idx_v.at[0, 0]], buf0, sem0)
        pltpu.async_copy(hp.at[idx_v.at[1, 0]], buf1, sem1)

        @pl.loop(0, SUPER, step=2)
        def _chunks(g):
            for b in range(2):
                j = g + b
                pltpu.make_async_copy(hp.at[idx_v.at[j, 0]], bufs[b], sems[b]).wait()
                pltpu.sync_copy(bufs[b], acc.at[idx_v.at[j, 1]], add=True)

                @pl.when(j + 2 < SUPER)
                def _next():
                    pltpu.async_copy(hp.at[idx_v.at[j + 2, 0]], bufs[b], sems[b])

    plsc.subcore_barrier()
    for k in range(RPS // CHUNK):
        pltpu.sync_copy(
            acc.at[pl.ds(s * RPS + k * CHUNK, CHUNK)],
            out.at[c].at[pl.ds(s * RPS + k * CHUNK, CHUNK)],
        )


def _tc_prep_body(deg_ref, x_ref, w_ref, hp_ref):
    cnt = deg_ref[0, :N, 0:1] + deg_ref[1, :N, 0:1] + 1.0
    dinv = lax.rsqrt(cnt)
    h = lax.dot_general(
        x_ref[...], w_ref[...], (((1,), (1,)), ((), ())),
        preferred_element_type=jnp.float32,
    )
    hp_ref[...] = h * dinv


_tc_prep = pl.pallas_call(
    _tc_prep_body, out_shape=jax.ShapeDtypeStruct((N, D), jnp.float32)
)


def _combine_bn_relu(acc_ref, hp_ref, deg_ref, b_ref, g_ref, be_ref):
    cnt = deg_ref[0, :N, 0:1] + deg_ref[1, :N, 0:1] + 1.0
    dinv = lax.rsqrt(cnt)
    ssum = acc_ref[0, :N, :] + acc_ref[1, :N, :] + hp_ref[...]
    z = ssum * dinv + b_ref[...]
    mu = jnp.mean(z, axis=0, keepdims=True)
    d = z - mu
    var = jnp.mean(d * d, axis=0, keepdims=True)
    h = d * lax.rsqrt(var + 1e-5) * g_ref[...] + be_ref[...]
    return jnp.maximum(h, 0.0), dinv


def _tc_mid_body(acc_ref, hp_ref, deg_ref, b_ref, g_ref, be_ref, w_ref, out_ref):
    h, dinv = _combine_bn_relu(acc_ref, hp_ref, deg_ref, b_ref, g_ref, be_ref)
    h2 = lax.dot_general(
        h, w_ref[...], (((1,), (1,)), ((), ())), preferred_element_type=jnp.float32
    )
    out_ref[...] = h2 * dinv


_tc_mid = pl.pallas_call(
    _tc_mid_body, out_shape=jax.ShapeDtypeStruct((N, D), jnp.float32)
)


def _tc_fin_body(acc_ref, hp_ref, deg_ref, b_ref, g_ref, be_ref, out_ref):
    h, _ = _combine_bn_relu(acc_ref, hp_ref, deg_ref, b_ref, g_ref, be_ref)
    out_ref[...] = h


_tc_fin = pl.pallas_call(
    _tc_fin_body, out_shape=jax.ShapeDtypeStruct((N, D), jnp.float32)
)


def kernel(x, edge_index, W1, b1, W2, b2, g1, be1, g2, be2):
    rows = edge_index[0]
    cols = edge_index[1]
    padlen = E_PAD - E
    rows_p = jnp.concatenate(
        [rows, jnp.zeros((padlen,), jnp.int32)]).reshape(CH_TOT, CHUNK)
    cols_p = jnp.concatenate(
        [cols, jnp.full((padlen,), N, jnp.int32)]).reshape(CH_TOT, CHUNK)
    edges_p = jnp.stack([rows_p, cols_p], axis=1)

    zeros_row = jnp.zeros((CHUNK, D), jnp.float32)

    deg = _sc_degree(edges_p)
    h1p = _tc_prep(deg, x, W1)
    acc1 = _sc_segsum(h1p, edges_p, zeros_row)
    b1r, g1r, be1r = b1.reshape(1, D), g1.reshape(1, D), be1.reshape(1, D)
    b2r, g2r, be2r = b2.reshape(1, D), g2.reshape(1, D), be2.reshape(1, D)
    h2p = _tc_mid(acc1, h1p, deg, b1r, g1r, be1r, W2)
    acc2 = _sc_segsum(h2p, edges_p, zeros_row)
    return _tc_fin(acc2, h2p, deg, b2r, g2r, be2r)


# speedup vs baseline: 1.2365x; 1.2365x over previous
import functools

import jax
import jax.numpy as jnp
from jax import lax
from jax.experimental import pallas as pl
from jax.experimental.pallas import tpu as pltpu
from jax.experimental.pallas import tpu_sc as plsc

N = 10000
D = 128
E = 320000

NC, NS = 2, 16
NW = NC * NS
CHUNK = 128
K_DEG = (80, 80)
K_SEG = (80, 80)
CH_TOT = NS * (K_SEG[0] + K_SEG[1])
E_PAD = CH_TOT * CHUNK
N_PAD = 10240
RPS = N_PAD // NS
SUPER = 16

_mesh = plsc.VectorSubcoreMesh(
    core_axis_name="c", subcore_axis_name="s", num_cores=NC, num_subcores=NS
)


def _core_span(c, s, k_pair):
    k0, k1 = k_pair
    base = jnp.where(c == 0, s * k0, NS * k0 + s * k1)
    nsup = jnp.where(c == 0, k0 // SUPER, k1 // SUPER)
    return base, nsup


def _fill(buf, value):
    v = jnp.full((16,), value, jnp.float32)

    @pl.loop(0, CHUNK)
    def _rows(r):
        for b in range(D // 16):
            buf[r, pl.ds(b * 16, 16)] = v


@functools.partial(
    pl.kernel,
    out_type=jax.ShapeDtypeStruct((NC, N_PAD, D), jnp.float32),
    mesh=_mesh,
    scratch_types=[
        pltpu.VMEM((SUPER, 2, CHUNK), jnp.int32),
        pltpu.VMEM((CHUNK, D), jnp.float32),
        pltpu.VMEM_SHARED((N_PAD, D), jnp.float32),
    ],
)
def _sc_degree(edges, out, idx_v, ones_v, acc):
    c = lax.axis_index("c")
    s = lax.axis_index("s")
    base, nsup = _core_span(c, s, K_DEG)
    _fill(ones_v, 0.0)
    for k in range(RPS // CHUNK):
        pltpu.sync_copy(ones_v, acc.at[pl.ds(s * RPS + k * CHUNK, CHUNK)])
    _fill(ones_v, 1.0)
    plsc.subcore_barrier()

    @pl.loop(0, nsup)
    def _super(sb):
        pltpu.sync_copy(edges.at[pl.ds(base + sb * SUPER, SUPER)], idx_v)

        @pl.loop(0, SUPER)
        def _count(j):
            pltpu.sync_copy(ones_v, acc.at[idx_v.at[j, 1]], add=True)

    plsc.subcore_barrier()
    for k in range(RPS // CHUNK):
        pltpu.sync_copy(
            acc.at[pl.ds(s * RPS + k * CHUNK, CHUNK)],
            out.at[c].at[pl.ds(s * RPS + k * CHUNK, CHUNK)],
        )


@functools.partial(
    pl.kernel,
    out_type=jax.ShapeDtypeStruct((NC, N_PAD, D), jnp.float32),
    mesh=_mesh,
    scratch_types=[
        pltpu.VMEM((SUPER, 2, CHUNK), jnp.int32),
        pltpu.VMEM((CHUNK, D), jnp.float32),
        pltpu.VMEM((CHUNK, D), jnp.float32),
        pltpu.VMEM_SHARED((N_PAD, D), jnp.float32),
        pltpu.SemaphoreType.DMA,
        pltpu.SemaphoreType.DMA,
    ],
)
def _sc_segsum(hp, edges, zeros_hbm, out, idx_v, buf0, buf1, acc, sem0, sem1):
    c = lax.axis_index("c")
    s = lax.axis_index("s")
    base, nsup = _core_span(c, s, K_SEG)
    pltpu.sync_copy(zeros_hbm, buf0)
    for k in range(RPS // CHUNK):
        pltpu.sync_copy(buf0, acc.at[pl.ds(s * RPS + k * CHUNK, CHUNK)])
    plsc.subcore_barrier()

    bufs = (buf0, buf1)
    sems = (sem0, sem1)

    @pl.loop(0, nsup)
    def _super(sb):
        pltpu.sync_copy(edges.at[pl.ds(base + sb * SUPER, SUPER)], idx_v)
        pltpu.async_copy(hp.at[idx_v.at[0, 0]], buf0, sem0)
        pltpu.async_copy(hp.at[idx_v.at[1, 0]], buf1, sem1)

        @pl.loop(0, SUPER, step=2)
        def _chunks(g):
            for b in range(2):
                j = g + b
                pltpu.make_async_copy(hp.at[idx_v.at[j, 0]], bufs[b], sems[b]).wait()
                pltpu.sync_copy(bufs[b], acc.at[idx_v.at[j, 1]], add=True)

                @pl.when(j + 2 < SUPER)
                def _next():
                    pltpu.async_copy(hp.at[idx_v.at[j + 2, 0]], bufs[b], sems[b])

    plsc.subcore_barrier()
    for k in range(RPS // CHUNK):
        pltpu.sync_copy(
            acc.at[pl.ds(s * RPS + k * CHUNK, CHUNK)],
            out.at[c].at[pl.ds(s * RPS + k * CHUNK, CHUNK)],
        )


def _tc_prep_body(deg_ref, x_ref, w_ref, hp_ref):
    cnt = deg_ref[0, :N, 0:1] + deg_ref[1, :N, 0:1] + 1.0
    dinv = lax.rsqrt(cnt)
    h = lax.dot_general(
        x_ref[...], w_ref[...], (((1,), (1,)), ((), ())),
        preferred_element_type=jnp.float32,
    )
    hp_ref[...] = h * dinv


_tc_prep = pl.pallas_call(
    _tc_prep_body, out_shape=jax.ShapeDtypeStruct((N, D), jnp.float32)
)


def _combine_bn_relu(acc_ref, hp_ref, deg_ref, b_ref, g_ref, be_ref):
    cnt = deg_ref[0, :N, 0:1] + deg_ref[1, :N, 0:1] + 1.0
    dinv = lax.rsqrt(cnt)
    ssum = acc_ref[0, :N, :] + acc_ref[1, :N, :] + hp_ref[...]
    z = ssum * dinv + b_ref[...]
    mu = jnp.mean(z, axis=0, keepdims=True)
    d = z - mu
    var = jnp.mean(d * d, axis=0, keepdims=True)
    h = d * lax.rsqrt(var + 1e-5) * g_ref[...] + be_ref[...]
    return jnp.maximum(h, 0.0), dinv


def _tc_mid_body(acc_ref, hp_ref, deg_ref, b_ref, g_ref, be_ref, w_ref, out_ref):
    h, dinv = _combine_bn_relu(acc_ref, hp_ref, deg_ref, b_ref, g_ref, be_ref)
    h2 = lax.dot_general(
        h, w_ref[...], (((1,), (1,)), ((), ())), preferred_element_type=jnp.float32
    )
    out_ref[...] = h2 * dinv


_tc_mid = pl.pallas_call(
    _tc_mid_body, out_shape=jax.ShapeDtypeStruct((N, D), jnp.float32)
)


def _tc_fin_body(acc_ref, hp_ref, deg_ref, b_ref, g_ref, be_ref, out_ref):
    h, _ = _combine_bn_relu(acc_ref, hp_ref, deg_ref, b_ref, g_ref, be_ref)
    out_ref[...] = h


_tc_fin = pl.pallas_call(
    _tc_fin_body, out_shape=jax.ShapeDtypeStruct((N, D), jnp.float32)
)


def kernel(x, edge_index, W1, b1, W2, b2, g1, be1, g2, be2):
    rows = edge_index[0]
    cols = edge_index[1]
    padlen = E_PAD - E
    pad_cols = N + jnp.arange(padlen, dtype=jnp.int32) % (N_PAD - N)
    rows_p = jnp.concatenate(
        [rows, jnp.zeros((padlen,), jnp.int32)]).reshape(CH_TOT, CHUNK)
    cols_p = jnp.concatenate([cols, pad_cols]).reshape(CH_TOT, CHUNK)
    edges_p = jnp.stack([rows_p, cols_p], axis=1)

    zeros_row = jnp.zeros((CHUNK, D), jnp.float32)

    deg = _sc_degree(edges_p)
    h1p = _tc_prep(deg, x, W1)
    acc1 = _sc_segsum(h1p, edges_p, zeros_row)
    b1r, g1r, be1r = b1.reshape(1, D), g1.reshape(1, D), be1.reshape(1, D)
    b2r, g2r, be2r = b2.reshape(1, D), g2.reshape(1, D), be2.reshape(1, D)
    h2p = _tc_mid(acc1, h1p, deg, b1r, g1r, be1r, W2)
    acc2 = _sc_segsum(h2p, edges_p, zeros_row)
    return _tc_fin(acc2, h2p, deg, b2r, g2r, be2r)
